# baseline (speedup 1.0000x reference)
import jax
import jax.numpy as jnp
from jax import lax
from jax.experimental import pallas as pl
from jax.experimental.pallas import tpu as pltpu
from jax.experimental.pallas import tpu_sc as plsc

B = 16384
D = 64
N = 1000000
NW = 32
BPW = B // NW
BLK = 16384

K_SC = 22
SC_N = K_SC * BLK
TC_N = N - SC_N
TC_GRID = (TC_N + BLK - 1) // BLK

CPW = SC_N // NW
CH = 256
NCHK = CPW // CH


def _tc_body(wu_ref, wm_ref, ut_ref, mt_ref, su_ref, sm_ref):
    su_ref[...] = jnp.dot(wu_ref[...], ut_ref[...],
                          preferred_element_type=jnp.float32)[0]
    sm_ref[...] = jnp.dot(wm_ref[...], mt_ref[...],
                          preferred_element_type=jnp.float32)[0]


def _tc_scan(ut_t, mt_t, wu, wm):
    return pl.pallas_call(
        _tc_body,
        grid=(TC_GRID,),
        in_specs=[
            pl.BlockSpec((1, D), lambda i: (0, 0)),
            pl.BlockSpec((1, D), lambda i: (0, 0)),
            pl.BlockSpec((D, BLK), lambda i: (0, K_SC + i)),
            pl.BlockSpec((D, BLK), lambda i: (0, K_SC + i)),
        ],
        out_specs=[
            pl.BlockSpec((BLK,), lambda i: (i,)),
            pl.BlockSpec((BLK,), lambda i: (i,)),
        ],
        out_shape=[
            jax.ShapeDtypeStruct((TC_N,), jnp.float32),
            jax.ShapeDtypeStruct((TC_N,), jnp.float32),
        ],
    )(wu, wm, ut_t, mt_t)


def _sc_scan_body(ut_hbm, mt_hbm, w_hbm, su_hbm, sm_hbm,
                  bu, bm, wv, svu, svm, sem):
    wid = lax.axis_index("s") * 2 + lax.axis_index("c")
    col0 = wid * CPW

    pltpu.sync_copy(w_hbm, wv)

    def fire(g, p):
        start = pl.multiple_of(col0 + g * CH, 128)
        pltpu.async_copy(ut_hbm.at[:, pl.ds(start, CH)], bu.at[p], sem)
        pltpu.async_copy(mt_hbm.at[:, pl.ds(start, CH)], bm.at[p], sem)

    fire(0, 0)

    def chunk(g, carry):
        p = lax.rem(g, 2)
        gn = lax.min(g + 1, NCHK - 1)
        fire(gn, 1 - p)
        pltpu.make_async_copy(ut_hbm.at[:, pl.ds(0, CH)], bu.at[p], sem).wait()
        pltpu.make_async_copy(mt_hbm.at[:, pl.ds(0, CH)], bm.at[p], sem).wait()

        def col_block(cv, c2):
            lsl = pl.ds(cv * 16, 16)
            accu = bu[p, 0, lsl] * wv[0, :]
            accm = bm[p, 0, lsl] * wv[D, :]
            for j in range(1, D):
                accu = accu + bu[p, j, lsl] * wv[j, :]
                accm = accm + bm[p, j, lsl] * wv[D + j, :]
            osl = pl.ds(g * CH + cv * 16, 16)
            svu[osl] = accu
            svm[osl] = accm
            return c2

        lax.fori_loop(0, CH // 16, col_block, 0)
        return carry

    lax.fori_loop(0, NCHK, chunk, 0)
    pltpu.make_async_copy(ut_hbm.at[:, pl.ds(0, CH)], bu.at[0], sem).wait()
    pltpu.make_async_copy(mt_hbm.at[:, pl.ds(0, CH)], bm.at[0], sem).wait()
    pltpu.sync_copy(svu, su_hbm.at[pl.ds(col0, CPW)])
    pltpu.sync_copy(svm, sm_hbm.at[pl.ds(col0, CPW)])


def _sc_scan(ut_t, mt_t, wflat):
    mesh = plsc.VectorSubcoreMesh(core_axis_name="c", subcore_axis_name="s")
    return pl.kernel(
        _sc_scan_body,
        out_type=[
            jax.ShapeDtypeStruct((SC_N,), jnp.float32),
            jax.ShapeDtypeStruct((SC_N,), jnp.float32),
        ],
        mesh=mesh,
        compiler_params=pltpu.CompilerParams(use_tc_tiling_on_sc=True),
        scratch_types=[
            pltpu.VMEM((2, D, CH), jnp.float32),
            pltpu.VMEM((2, D, CH), jnp.float32),
            pltpu.VMEM((2 * D, 16), jnp.float32),
            pltpu.VMEM((CPW,), jnp.float32),
            pltpu.VMEM((CPW,), jnp.float32),
            pltpu.SemaphoreType.DMA,
        ],
    )(ut_t, mt_t, wflat)


def _sc_gather_body(uid_hbm, mid_hbm, ssu_hbm, ssm_hbm, stu_hbm, stm_hbm,
                    bias_hbm, out_hbm,
                    idx_u, idx_m, isu, itu, ism, itm,
                    gsu, gtu, gsm, gtm, bv, outv, sem):
    wid = lax.axis_index("s") * 2 + lax.axis_index("c")
    base = wid * BPW

    pltpu.sync_copy(uid_hbm.at[pl.ds(base, BPW)], idx_u)
    pltpu.sync_copy(mid_hbm.at[pl.ds(base, BPW)], idx_m)
    pltpu.sync_copy(bias_hbm, bv)

    def split(g, carry):
        sl = pl.ds(g * 16, 16)
        iu = idx_u[sl]
        im = idx_m[sl]
        zero = jnp.zeros((16,), jnp.int32)
        isu[sl] = jnp.where(iu < SC_N, iu, zero)
        itu[sl] = jnp.where(iu >= SC_N, iu - SC_N, zero)
        ism[sl] = jnp.where(im < SC_N, im, zero)
        itm[sl] = jnp.where(im >= SC_N, im - SC_N, zero)
        return carry

    lax.fori_loop(0, BPW // 16, split, 0)

    copies = []
    for j in range(BPW // 128):
        sl = pl.ds(j * 128, 128)
        copies.append(pltpu.async_copy(ssu_hbm.at[isu.at[sl]], gsu.at[sl], sem))
        copies.append(pltpu.async_copy(stu_hbm.at[itu.at[sl]], gtu.at[sl], sem))
        copies.append(pltpu.async_copy(ssm_hbm.at[ism.at[sl]], gsm.at[sl], sem))
        copies.append(pltpu.async_copy(stm_hbm.at[itm.at[sl]], gtm.at[sl], sem))
    for c in copies:
        c.wait()

    bvec = bv[pl.ds(0, 16)]

    def combine(g, carry):
        sl = pl.ds(g * 16, 16)
        iu = idx_u[sl]
        im = idx_m[sl]
        vu = jnp.where(iu < SC_N, gsu[sl], gtu[sl])
        vm = jnp.where(im < SC_N, gsm[sl], gtm[sl])
        outv[sl] = vu + vm + bvec
        return carry

    lax.fori_loop(0, BPW // 16, combine, 0)
    pltpu.sync_copy(outv, out_hbm.at[pl.ds(base, BPW)])


def _sc_gather(uid, mid, ssu, ssm, stu, stm, bias16):
    mesh = plsc.VectorSubcoreMesh(core_axis_name="c", subcore_axis_name="s")
    return pl.kernel(
        _sc_gather_body,
        out_type=jax.ShapeDtypeStruct((B,), jnp.float32),
        mesh=mesh,
        scratch_types=[
            pltpu.VMEM((BPW,), jnp.int32),
            pltpu.VMEM((BPW,), jnp.int32),
            pltpu.VMEM((BPW,), jnp.int32),
            pltpu.VMEM((BPW,), jnp.int32),
            pltpu.VMEM((BPW,), jnp.int32),
            pltpu.VMEM((BPW,), jnp.int32),
            pltpu.VMEM((BPW,), jnp.float32),
            pltpu.VMEM((BPW,), jnp.float32),
            pltpu.VMEM((BPW,), jnp.float32),
            pltpu.VMEM((BPW,), jnp.float32),
            pltpu.VMEM((16,), jnp.float32),
            pltpu.VMEM((BPW,), jnp.float32),
            pltpu.SemaphoreType.DMA,
        ],
    )(uid, mid, ssu, ssm, stu, stm, bias16)


@jax.jit
def kernel(user_id, movie_id, user_table, movie_table, W, b):
    uid = user_id.astype(jnp.int32)
    mid = movie_id.astype(jnp.int32)
    ut_t = user_table.T
    mt_t = movie_table.T
    wu = W[:D, 0][None, :]
    wm = W[D:, 0][None, :]
    wexp = jnp.repeat(W[:, 0:1], 16, axis=1)
    ssu, ssm = _sc_scan(ut_t, mt_t, wexp)
    stu, stm = _tc_scan(ut_t, mt_t, wu, wm)
    bias16 = jnp.full((16,), b[0], jnp.float32)
    out = _sc_gather(uid, mid, ssu, ssm, stu, stm, bias16)
    return out[:, None]


# speedup vs baseline: 1.2184x; 1.2184x over previous
import jax
import jax.numpy as jnp
from jax import lax
from jax.experimental import pallas as pl
from jax.experimental.pallas import tpu as pltpu
from jax.experimental.pallas import tpu_sc as plsc

B = 16384
D = 64
N = 1000000
NW = 32
BPW = B // NW
BLK = 16384

K_SC = 18
SC_N = K_SC * BLK
TC_N = N - SC_N
TC_GRID = (TC_N + BLK - 1) // BLK

CPW = SC_N // NW
CH = 256
NCHK = CPW // CH


def _tc_body(wu_ref, wm_ref, ut_ref, mt_ref, su_ref, sm_ref):
    su_ref[...] = jnp.dot(wu_ref[...], ut_ref[...],
                          preferred_element_type=jnp.float32)[0]
    sm_ref[...] = jnp.dot(wm_ref[...], mt_ref[...],
                          preferred_element_type=jnp.float32)[0]


def _tc_scan(ut_t, mt_t, wu, wm):
    return pl.pallas_call(
        _tc_body,
        grid=(TC_GRID,),
        in_specs=[
            pl.BlockSpec((1, D), lambda i: (0, 0)),
            pl.BlockSpec((1, D), lambda i: (0, 0)),
            pl.BlockSpec((D, BLK), lambda i: (0, K_SC + i)),
            pl.BlockSpec((D, BLK), lambda i: (0, K_SC + i)),
        ],
        out_specs=[
            pl.BlockSpec((BLK,), lambda i: (i,)),
            pl.BlockSpec((BLK,), lambda i: (i,)),
        ],
        out_shape=[
            jax.ShapeDtypeStruct((TC_N,), jnp.float32),
            jax.ShapeDtypeStruct((TC_N,), jnp.float32),
        ],
    )(wu, wm, ut_t, mt_t)


def _sc_scan_body(ut_hbm, mt_hbm, w_hbm, su_hbm, sm_hbm,
                  bu, bm, wv, svu, svm, sem):
    wid = lax.axis_index("s") * 2 + lax.axis_index("c")
    col0 = wid * CPW

    pltpu.sync_copy(w_hbm, wv)

    def fire(g, p):
        start = pl.multiple_of(col0 + g * CH, 128)
        pltpu.async_copy(ut_hbm.at[:, pl.ds(start, CH)], bu.at[p], sem)
        pltpu.async_copy(mt_hbm.at[:, pl.ds(start, CH)], bm.at[p], sem)

    fire(0, 0)

    def chunk(g, carry):
        p = lax.rem(g, 2)
        gn = lax.min(g + 1, NCHK - 1)
        fire(gn, 1 - p)
        pltpu.make_async_copy(ut_hbm.at[:, pl.ds(0, CH)], bu.at[p], sem).wait()
        pltpu.make_async_copy(mt_hbm.at[:, pl.ds(0, CH)], bm.at[p], sem).wait()

        def col_block(cv, c2):
            lsl = pl.ds(cv * 16, 16)
            accu = bu[p, 0, lsl] * wv[0, :]
            accm = bm[p, 0, lsl] * wv[D, :]
            for j in range(1, D):
                accu = accu + bu[p, j, lsl] * wv[j, :]
                accm = accm + bm[p, j, lsl] * wv[D + j, :]
            osl = pl.ds(g * CH + cv * 16, 16)
            svu[osl] = accu
            svm[osl] = accm
            return c2

        lax.fori_loop(0, CH // 16, col_block, 0)
        return carry

    lax.fori_loop(0, NCHK, chunk, 0)
    pltpu.make_async_copy(ut_hbm.at[:, pl.ds(0, CH)], bu.at[0], sem).wait()
    pltpu.make_async_copy(mt_hbm.at[:, pl.ds(0, CH)], bm.at[0], sem).wait()
    pltpu.sync_copy(svu, su_hbm.at[pl.ds(col0, CPW)])
    pltpu.sync_copy(svm, sm_hbm.at[pl.ds(col0, CPW)])


def _sc_scan(ut_t, mt_t, wflat):
    mesh = plsc.VectorSubcoreMesh(core_axis_name="c", subcore_axis_name="s")
    return pl.kernel(
        _sc_scan_body,
        out_type=[
            jax.ShapeDtypeStruct((SC_N,), jnp.float32),
            jax.ShapeDtypeStruct((SC_N,), jnp.float32),
        ],
        mesh=mesh,
        compiler_params=pltpu.CompilerParams(use_tc_tiling_on_sc=True),
        scratch_types=[
            pltpu.VMEM((2, D, CH), jnp.float32),
            pltpu.VMEM((2, D, CH), jnp.float32),
            pltpu.VMEM((2 * D, 16), jnp.float32),
            pltpu.VMEM((CPW,), jnp.float32),
            pltpu.VMEM((CPW,), jnp.float32),
            pltpu.SemaphoreType.DMA,
        ],
    )(ut_t, mt_t, wflat)


def _sc_gather_body(uid_hbm, mid_hbm, su_hbm, sm_hbm, bias_hbm, out_hbm,
                    idx_u, idx_m, g_u, g_m, bv, outv, sem):
    wid = lax.axis_index("s") * 2 + lax.axis_index("c")
    base = wid * BPW

    pltpu.sync_copy(uid_hbm.at[pl.ds(base, BPW)], idx_u)
    pltpu.sync_copy(mid_hbm.at[pl.ds(base, BPW)], idx_m)
    pltpu.sync_copy(bias_hbm, bv)

    copies = []
    for j in range(BPW // 128):
        sl = pl.ds(j * 128, 128)
        copies.append(pltpu.async_copy(su_hbm.at[idx_u.at[sl]], g_u.at[sl], sem))
        copies.append(pltpu.async_copy(sm_hbm.at[idx_m.at[sl]], g_m.at[sl], sem))
    for c in copies:
        c.wait()

    bvec = bv[pl.ds(0, 16)]

    def body(g, carry):
        sl = pl.ds(g * 16, 16)
        outv[sl] = g_u[sl] + g_m[sl] + bvec
        return carry

    lax.fori_loop(0, BPW // 16, body, 0)
    pltpu.sync_copy(outv, out_hbm.at[pl.ds(base, BPW)])


def _sc_gather(uid, mid, su, sm, bias16):
    mesh = plsc.VectorSubcoreMesh(core_axis_name="c", subcore_axis_name="s")
    return pl.kernel(
        _sc_gather_body,
        out_type=jax.ShapeDtypeStruct((B,), jnp.float32),
        mesh=mesh,
        scratch_types=[
            pltpu.VMEM((BPW,), jnp.int32),
            pltpu.VMEM((BPW,), jnp.int32),
            pltpu.VMEM((BPW,), jnp.float32),
            pltpu.VMEM((BPW,), jnp.float32),
            pltpu.VMEM((16,), jnp.float32),
            pltpu.VMEM((BPW,), jnp.float32),
            pltpu.SemaphoreType.DMA,
        ],
    )(uid, mid, su, sm, bias16)


@jax.jit
def kernel(user_id, movie_id, user_table, movie_table, W, b):
    uid = user_id.astype(jnp.int32)
    mid = movie_id.astype(jnp.int32)
    ut_t = user_table.T
    mt_t = movie_table.T
    wu = W[:D, 0][None, :]
    wm = W[D:, 0][None, :]
    wexp = jnp.repeat(W[:, 0:1], 16, axis=1)
    ssu, ssm = _sc_scan(ut_t, mt_t, wexp)
    stu, stm = _tc_scan(ut_t, mt_t, wu, wm)
    su = jnp.concatenate([ssu, stu])
    sm = jnp.concatenate([ssm, stm])
    bias16 = jnp.full((16,), b[0], jnp.float32)
    out = _sc_gather(uid, mid, su, sm, bias16)
    return out[:, None]


# speedup vs baseline: 1.3202x; 1.0835x over previous
import functools
import jax
import jax.numpy as jnp
from jax import lax
from jax.experimental import pallas as pl
from jax.experimental.pallas import tpu as pltpu
from jax.experimental.pallas import tpu_sc as plsc

B = 16384
D = 64
N = 1000000
NW = 32
BPW = B // NW
BLK = 20480
GRID = (N + BLK - 1) // BLK


def _tc_body(wu_ref, wm_ref, ut_ref, mt_ref, su_ref, sm_ref):
    su_ref[...] = jnp.dot(wu_ref[...], ut_ref[...],
                          preferred_element_type=jnp.float32)[0]
    sm_ref[...] = jnp.dot(wm_ref[...], mt_ref[...],
                          preferred_element_type=jnp.float32)[0]


def _tc_scan(ut_t, mt_t, wu, wm):
    return pl.pallas_call(
        _tc_body,
        grid=(GRID,),
        in_specs=[
            pl.BlockSpec((1, D), lambda i: (0, 0)),
            pl.BlockSpec((1, D), lambda i: (0, 0)),
            pl.BlockSpec((D, BLK), lambda i: (0, i)),
            pl.BlockSpec((D, BLK), lambda i: (0, i)),
        ],
        out_specs=[
            pl.BlockSpec((BLK,), lambda i: (i,)),
            pl.BlockSpec((BLK,), lambda i: (i,)),
        ],
        out_shape=[
            jax.ShapeDtypeStruct((N,), jnp.float32),
            jax.ShapeDtypeStruct((N,), jnp.float32),
        ],
    )(wu, wm, ut_t, mt_t)


def _sc_body(uid_hbm, mid_hbm, su_hbm, sm_hbm, bias_hbm, out_hbm,
             idx_u, idx_m, g_u, g_m, bv, outv, sem):
    wid = lax.axis_index("s") * 2 + lax.axis_index("c")
    base = wid * BPW

    pltpu.sync_copy(uid_hbm.at[pl.ds(base, BPW)], idx_u)
    pltpu.sync_copy(mid_hbm.at[pl.ds(base, BPW)], idx_m)
    pltpu.sync_copy(bias_hbm, bv)

    copies = []
    for j in range(BPW // 128):
        sl = pl.ds(j * 128, 128)
        copies.append(pltpu.async_copy(su_hbm.at[idx_u.at[sl]], g_u.at[sl], sem))
        copies.append(pltpu.async_copy(sm_hbm.at[idx_m.at[sl]], g_m.at[sl], sem))
    for c in copies:
        c.wait()

    bvec = bv[pl.ds(0, 16)]

    def body(g, carry):
        sl = pl.ds(g * 16, 16)
        outv[sl] = g_u[sl] + g_m[sl] + bvec
        return carry

    lax.fori_loop(0, BPW // 16, body, 0)
    pltpu.sync_copy(outv, out_hbm.at[pl.ds(base, BPW)])


def _sc_gather(uid, mid, su, sm, bias16):
    mesh = plsc.VectorSubcoreMesh(core_axis_name="c", subcore_axis_name="s")
    return pl.kernel(
        _sc_body,
        out_type=jax.ShapeDtypeStruct((B,), jnp.float32),
        mesh=mesh,
        scratch_types=[
            pltpu.VMEM((BPW,), jnp.int32),
            pltpu.VMEM((BPW,), jnp.int32),
            pltpu.VMEM((BPW,), jnp.float32),
            pltpu.VMEM((BPW,), jnp.float32),
            pltpu.VMEM((16,), jnp.float32),
            pltpu.VMEM((BPW,), jnp.float32),
            pltpu.SemaphoreType.DMA,
        ],
    )(uid, mid, su, sm, bias16)


@jax.jit
def kernel(user_id, movie_id, user_table, movie_table, W, b):
    uid = user_id.astype(jnp.int32)
    mid = movie_id.astype(jnp.int32)
    wu = W[:D, 0][None, :]
    wm = W[D:, 0][None, :]
    su, sm = _tc_scan(user_table.T, movie_table.T, wu, wm)
    bias16 = jnp.full((16,), b[0], jnp.float32)
    out = _sc_gather(uid, mid, su, sm, bias16)
    return out[:, None]
